# baseline (speedup 1.0000x reference)
import jax
import jax.numpy as jnp
from jax.experimental import pallas as pl

D_MODEL = 64
N_BITS_PAD = 32
BLOCK = 16384


def _body(x_ref, t_ref, o_ref):
    xrow = x_ref[0]
    iot = jax.lax.broadcasted_iota(jnp.int32, (N_BITS_PAD, 1), 0)
    bits_t = ((xrow >> iot) & 1).astype(jnp.float32)
    o_ref[0] = jax.lax.dot_general(
        bits_t,
        t_ref[...],
        (((0,), (0,)), ((), ())),
        preferred_element_type=jnp.float32,
    )


def kernel(x, table):
    x_shape = x.shape
    n = x.size
    assert n % BLOCK == 0, n
    nb = n // BLOCK
    xf = x.reshape(nb, 1, BLOCK)
    tpad = jnp.zeros((N_BITS_PAD, D_MODEL), table.dtype).at[: table.shape[0]].set(table)
    out = pl.pallas_call(
        _body,
        grid=(nb,),
        in_specs=[
            pl.BlockSpec((1, 1, BLOCK), lambda i: (i, 0, 0)),
            pl.BlockSpec((N_BITS_PAD, D_MODEL), lambda i: (0, 0)),
        ],
        out_specs=pl.BlockSpec((1, BLOCK, D_MODEL), lambda i: (i, 0, 0)),
        out_shape=jax.ShapeDtypeStruct((nb, BLOCK, D_MODEL), jnp.float32),
    )(xf, tpad)
    return out.reshape(*x_shape, D_MODEL)
